# baseline (speedup 1.0000x reference)
import functools

import jax
import jax.numpy as jnp
from jax import lax
from jax.experimental import pallas as pl
from jax.experimental.pallas import tpu as pltpu
from jax.experimental.pallas import tpu_sc as plsc

_L = 16
_NBIN = 256
_UNROLL = 8
_HIST_WORDS = -(-(16 * 257) // (16 * _UNROLL)) * (16 * _UNROLL)


def _logits_body(x_ref, w_ref, out_ref):
    xb = x_ref[0]
    w = w_ref[...]
    out_ref[0] = jax.lax.dot_general(
        w, xb, (((1,), (1,)), ((), ())),
        preferred_element_type=jnp.float32)


def _sc_sort_body(nbatch, t, logit_hbm, sig_hbm, idx_hbm,
                  lg, ka, kb, pa, pb, hist, hist2, tot, dbase, sg):
    lane = lax.iota(jnp.int32, _L)
    per_lane = t // _L
    _PL = per_lane + 1
    lane_blk = lane * _PL
    lane_hist = lane * (_NBIN + 1)
    zero16 = jnp.zeros((_L,), jnp.int32)
    ones16 = jnp.ones((_L,), jnp.int32)
    nvec = t // _L

    wid = lax.axis_index("s") * 2 + lax.axis_index("c")

    @pl.when(wid < nbatch)
    def _():
        row = wid * t
        pltpu.sync_copy(logit_hbm.at[pl.ds(row, t)], lg)

        def mk(jo, _):
            for u in range(_UNROLL):
                seq = (jo * _UNROLL + u) * _L + lane
                padseq = seq + lax.shift_right_logical(seq, 8)
                v = plsc.bitcast(plsc.load_gather(lg, [seq]), jnp.int32)
                m = lax.shift_right_arithmetic(v, 31)
                k = v ^ ((m ^ jnp.int32(-1)) & jnp.int32(0x7FFFFFFF))
                plsc.store_scatter(ka, [padseq], k)
                plsc.store_scatter(pa, [padseq], seq)
            return 0
        lax.fori_loop(0, nvec // _UNROLL, mk, 0)

        hists = (hist, hist2)
        sub = per_lane // len(hists)

        def radix_pass(src_k, src_p, dst_k, dst_p, shift, last=False):
            def z(jo, _):
                for u in range(_UNROLL):
                    idxs = (jo * _UNROLL + u) * _L + lane
                    for hs in hists:
                        plsc.store_scatter(hs, [idxs], zero16)
                return 0
            lax.fori_loop(0, _HIST_WORDS // (_L * _UNROLL), z, 0)

            def h(jo, _):
                for u in range(_UNROLL // 2):
                    j = jo * (_UNROLL // 2) + u
                    for s, hs in enumerate(hists):
                        k = plsc.load_gather(src_k, [lane_blk + s * sub + j])
                        d = lax.shift_right_logical(k, shift) & 255
                        plsc.addupdate_scatter(hs, [lane_hist + d], ones16)
                return 0
            lax.fori_loop(0, sub // (_UNROLL // 2), h, 0)

            def b1(c, _):
                dchunk = c * _L + lane
                s = zero16
                for l in range(_L):
                    base_i = l * (_NBIN + 1) + dchunk
                    c0 = plsc.load_gather(hist, [base_i])
                    c1 = plsc.load_gather(hist2, [base_i])
                    plsc.store_scatter(hist, [base_i], s)
                    plsc.store_scatter(hist2, [base_i], s + c0)
                    s = s + c0 + c1
                plsc.store_scatter(tot, [dchunk], s)
                return 0
            lax.fori_loop(0, _NBIN // _L, b1, 0)

            def b2(c, carry):
                v = plsc.load_gather(tot, [c * _L + lane])
                incl = plsc.cumsum(v)
                plsc.store_scatter(dbase, [c * _L + lane],
                                   (incl - v) + carry)
                return carry + jnp.sum(v)
            lax.fori_loop(0, _NBIN // _L, b2, jnp.int32(0))

            def b3(c, _):
                dchunk = c * _L + lane
                bv = plsc.load_gather(dbase, [dchunk])
                for l in range(_L):
                    base_i = l * (_NBIN + 1) + dchunk
                    p0 = plsc.load_gather(hist, [base_i])
                    p1 = plsc.load_gather(hist2, [base_i])
                    plsc.store_scatter(hist, [base_i], p0 + bv)
                    plsc.store_scatter(hist2, [base_i], p1 + bv)
                return 0
            lax.fori_loop(0, _NBIN // _L, b3, 0)

            def sc(jo, _):
                for u in range(_UNROLL // 2):
                    j = jo * (_UNROLL // 2) + u
                    for s, hs in enumerate(hists):
                        k = plsc.load_gather(src_k, [lane_blk + s * sub + j])
                        p = plsc.load_gather(src_p, [lane_blk + s * sub + j])
                        d = lax.shift_right_logical(k, shift) & 255
                        hidx = lane_hist + d
                        dest = plsc.load_gather(hs, [hidx])
                        plsc.addupdate_scatter(hs, [hidx], ones16)
                        if last:
                            plsc.store_scatter(dst_p, [dest], p)
                        else:
                            pdest = dest + lax.shift_right_logical(dest, 8)
                            plsc.store_scatter(dst_k, [pdest], k)
                            plsc.store_scatter(dst_p, [pdest], p)
                return 0
            lax.fori_loop(0, sub // (_UNROLL // 2), sc, 0)

        radix_pass(ka, pa, kb, pb, 0)
        radix_pass(kb, pb, ka, pa, 8)
        radix_pass(ka, pa, kb, pb, 16)
        radix_pass(kb, pb, ka, pa, 24, last=True)

        def fin(jo, _):
            for u in range(_UNROLL):
                seq = (jo * _UNROLL + u) * _L + lane
                pidx = plsc.load_gather(pa, [seq])
                v = plsc.load_gather(lg, [pidx])
                s = 1.0 / (1.0 + jnp.exp(-v))
                plsc.store_scatter(sg, [seq], s)
            return 0
        lax.fori_loop(0, nvec // _UNROLL, fin, 0)

        pltpu.sync_copy(sg, sig_hbm.at[pl.ds(row, t)])
        pltpu.sync_copy(pa.at[pl.ds(0, t)], idx_hbm.at[pl.ds(row, t)])


def kernel(x, W, capacity_factor):
    B, T, C = x.shape
    TT = min(2048, T)

    logits = pl.pallas_call(
        _logits_body,
        grid=(B, T // TT),
        in_specs=[
            pl.BlockSpec((1, TT, C), lambda b, t: (b, t, 0)),
            pl.BlockSpec((1, C), lambda b, t: (0, 0)),
        ],
        out_specs=pl.BlockSpec((1, 1, TT), lambda b, t: (b, 0, t)),
        out_shape=jax.ShapeDtypeStruct((B, 1, T), jnp.float32),
    )(x, W)

    mesh = plsc.VectorSubcoreMesh(core_axis_name="c", subcore_axis_name="s")
    sc_sort = functools.partial(
        pl.kernel,
        mesh=mesh,
        out_type=[
            jax.ShapeDtypeStruct((B * T,), jnp.float32),
            jax.ShapeDtypeStruct((B * T,), jnp.int32),
        ],
        scratch_types=[
            pltpu.VMEM((T,), jnp.float32),
            pltpu.VMEM((T + _L,), jnp.int32),
            pltpu.VMEM((T + _L,), jnp.int32),
            pltpu.VMEM((T + _L,), jnp.int32),
            pltpu.VMEM((T + _L,), jnp.int32),
            pltpu.VMEM((_HIST_WORDS,), jnp.int32),
            pltpu.VMEM((_HIST_WORDS,), jnp.int32),
            pltpu.VMEM((_NBIN,), jnp.int32),
            pltpu.VMEM((_NBIN,), jnp.int32),
            pltpu.VMEM((T,), jnp.float32),
        ],
        compiler_params=pltpu.CompilerParams(
            use_tc_tiling_on_sc=False, needs_layout_passes=False),
    )(functools.partial(_sc_sort_body, B, T))

    sig_flat, idx_flat = sc_sort(logits.reshape(B * T))

    weights = sig_flat.reshape(B, T, 1)
    selected_tokens = idx_flat.reshape(B, T, 1)
    is_final = jnp.zeros((B, T), dtype=bool)
    return (is_final, selected_tokens, weights)


# speedup vs baseline: 1.0388x; 1.0388x over previous
import functools

import jax
import jax.numpy as jnp
from jax import lax
from jax.experimental import pallas as pl
from jax.experimental.pallas import tpu as pltpu
from jax.experimental.pallas import tpu_sc as plsc

_L = 16
_NBIN = 256
_UNROLL = 8
_HIST_WORDS = -(-(16 * 257) // (16 * _UNROLL)) * (16 * _UNROLL)


def _logits_body(x_ref, w_ref, out_ref):
    xb = x_ref[0]
    w = w_ref[...]
    out_ref[0] = jax.lax.dot_general(
        w, xb, (((1,), (1,)), ((), ())),
        preferred_element_type=jnp.float32)


def _sc_sort_body(nbatch, t, logit_hbm, sig_hbm, idx_hbm,
                  lg, ka, kb, pa, pb, hist, tot, dbase, sg):
    lane = lax.iota(jnp.int32, _L)
    per_lane = t // _L
    _PL = per_lane + 1
    lane_blk = lane * _PL
    lane_hist = lane * (_NBIN + 1)
    zero16 = jnp.zeros((_L,), jnp.int32)
    ones16 = jnp.ones((_L,), jnp.int32)
    nvec = t // _L

    wid = lax.axis_index("s") * 2 + lax.axis_index("c")

    @pl.when(wid < nbatch)
    def _():
        row = wid * t
        pltpu.sync_copy(logit_hbm.at[pl.ds(row, t)], lg)

        def mk(jo, _):
            for u in range(_UNROLL):
                seq = (jo * _UNROLL + u) * _L + lane
                padseq = seq + lax.shift_right_logical(seq, 8)
                v = plsc.bitcast(plsc.load_gather(lg, [seq]), jnp.int32)
                m = lax.shift_right_arithmetic(v, 31)
                k = v ^ ((m ^ jnp.int32(-1)) & jnp.int32(0x7FFFFFFF))
                plsc.store_scatter(ka, [padseq], k)
                plsc.store_scatter(pa, [padseq], seq)
            return 0
        lax.fori_loop(0, nvec // _UNROLL, mk, 0)

        def radix_pass(src_k, src_p, dst_k, dst_p, shift, last=False):
            def z(jo, _):
                for u in range(_UNROLL):
                    plsc.store_scatter(
                        hist, [(jo * _UNROLL + u) * _L + lane], zero16)
                return 0
            lax.fori_loop(0, _HIST_WORDS // (_L * _UNROLL), z, 0)

            def h(jo, _):
                for u in range(_UNROLL):
                    j = jo * _UNROLL + u
                    k = plsc.load_gather(src_k, [lane_blk + j])
                    d = lax.shift_right_logical(k, shift) & 255
                    plsc.addupdate_scatter(hist, [lane_hist + d], ones16)
                return 0
            lax.fori_loop(0, nvec // _UNROLL, h, 0)

            def b1(c, _):
                dchunk = c * _L + lane
                s = zero16
                for l in range(_L):
                    hl = plsc.load_gather(hist, [l * (_NBIN + 1) + dchunk])
                    plsc.store_scatter(hist, [l * (_NBIN + 1) + dchunk], s)
                    s = s + hl
                plsc.store_scatter(tot, [dchunk], s)
                return 0
            lax.fori_loop(0, _NBIN // _L, b1, 0)

            def b2(c, carry):
                v = plsc.load_gather(tot, [c * _L + lane])
                incl = plsc.cumsum(v)
                plsc.store_scatter(dbase, [c * _L + lane],
                                   (incl - v) + carry)
                return carry + jnp.sum(v)
            lax.fori_loop(0, _NBIN // _L, b2, jnp.int32(0))

            def b3(c, _):
                dchunk = c * _L + lane
                bv = plsc.load_gather(dbase, [dchunk])
                for l in range(_L):
                    p = plsc.load_gather(hist, [l * (_NBIN + 1) + dchunk])
                    plsc.store_scatter(hist, [l * (_NBIN + 1) + dchunk],
                                       p + bv)
                return 0
            lax.fori_loop(0, _NBIN // _L, b3, 0)

            def sc(jo, _):
                for u in range(_UNROLL):
                    j = jo * _UNROLL + u
                    k = plsc.load_gather(src_k, [lane_blk + j])
                    p = plsc.load_gather(src_p, [lane_blk + j])
                    d = lax.shift_right_logical(k, shift) & 255
                    hidx = lane_hist + d
                    dest = plsc.load_gather(hist, [hidx])
                    plsc.addupdate_scatter(hist, [hidx], ones16)
                    if last:
                        plsc.store_scatter(dst_p, [dest], p)
                    else:
                        pdest = dest + lax.shift_right_logical(dest, 8)
                        plsc.store_scatter(dst_k, [pdest], k)
                        plsc.store_scatter(dst_p, [pdest], p)
                return 0
            lax.fori_loop(0, nvec // _UNROLL, sc, 0)

        radix_pass(ka, pa, kb, pb, 0)
        radix_pass(kb, pb, ka, pa, 8)
        radix_pass(ka, pa, kb, pb, 16)
        radix_pass(kb, pb, ka, pa, 24, last=True)

        def fin(jo, _):
            for u in range(_UNROLL):
                seq = (jo * _UNROLL + u) * _L + lane
                pidx = plsc.load_gather(pa, [seq])
                v = plsc.load_gather(lg, [pidx])
                s = 1.0 / (1.0 + jnp.exp(-v))
                plsc.store_scatter(sg, [seq], s)
            return 0
        lax.fori_loop(0, nvec // _UNROLL, fin, 0)

        pltpu.sync_copy(sg, sig_hbm.at[pl.ds(row, t)])
        pltpu.sync_copy(pa.at[pl.ds(0, t)], idx_hbm.at[pl.ds(row, t)])


def kernel(x, W, capacity_factor):
    B, T, C = x.shape
    TT = min(2048, T)

    logits = pl.pallas_call(
        _logits_body,
        grid=(B, T // TT),
        in_specs=[
            pl.BlockSpec((1, TT, C), lambda b, t: (b, t, 0)),
            pl.BlockSpec((1, C), lambda b, t: (0, 0)),
        ],
        out_specs=pl.BlockSpec((1, 1, TT), lambda b, t: (b, 0, t)),
        out_shape=jax.ShapeDtypeStruct((B, 1, T), jnp.float32),
    )(x, W)

    mesh = plsc.VectorSubcoreMesh(core_axis_name="c", subcore_axis_name="s")
    sc_sort = functools.partial(
        pl.kernel,
        mesh=mesh,
        out_type=[
            jax.ShapeDtypeStruct((B * T,), jnp.float32),
            jax.ShapeDtypeStruct((B * T,), jnp.int32),
        ],
        scratch_types=[
            pltpu.VMEM((T,), jnp.float32),
            pltpu.VMEM((T + _L,), jnp.int32),
            pltpu.VMEM((T + _L,), jnp.int32),
            pltpu.VMEM((T + _L,), jnp.int32),
            pltpu.VMEM((T + _L,), jnp.int32),
            pltpu.VMEM((_HIST_WORDS,), jnp.int32),
            pltpu.VMEM((_NBIN,), jnp.int32),
            pltpu.VMEM((_NBIN,), jnp.int32),
            pltpu.VMEM((T,), jnp.float32),
        ],
        compiler_params=pltpu.CompilerParams(
            use_tc_tiling_on_sc=False, needs_layout_passes=False),
    )(functools.partial(_sc_sort_body, B, T))

    sig_flat, idx_flat = sc_sort(logits.reshape(B * T))

    weights = sig_flat.reshape(B, T, 1)
    selected_tokens = idx_flat.reshape(B, T, 1)
    is_final = jnp.zeros((B, T), dtype=bool)
    return (is_final, selected_tokens, weights)


# speedup vs baseline: 1.0808x; 1.0404x over previous
import functools

import jax
import jax.numpy as jnp
from jax import lax
from jax.experimental import pallas as pl
from jax.experimental.pallas import tpu as pltpu
from jax.experimental.pallas import tpu_sc as plsc

_L = 16
_NBIN = 256
_UNROLL = 8
_HIST_WORDS = -(-(16 * 257) // (16 * _UNROLL)) * (16 * _UNROLL)


def _logits_body(x_ref, w_ref, out_ref):
    xb = x_ref[0]
    w = w_ref[...]
    out_ref[0] = jax.lax.dot_general(
        w, xb, (((1,), (1,)), ((), ())),
        preferred_element_type=jnp.float32)


def _sc_sort_body(nbatch, t, logit_hbm, sig_hbm, idx_hbm,
                  lg, ka, kb, pa, pb, hist, tot, dbase, sg):
    lane = lax.iota(jnp.int32, _L)
    per_lane = t // _L
    _PL = per_lane + 1
    lane_blk = lane * _PL
    lane_hist = lane * (_NBIN + 1)
    zero16 = jnp.zeros((_L,), jnp.int32)
    ones16 = jnp.ones((_L,), jnp.int32)
    nvec = t // _L

    wid = lax.axis_index("s") * 2 + lax.axis_index("c")

    @pl.when(wid < nbatch)
    def _():
        row = wid * t
        pltpu.sync_copy(logit_hbm.at[pl.ds(row, t)], lg)

        def mk(jo, _):
            for u in range(_UNROLL):
                seq = (jo * _UNROLL + u) * _L + lane
                padseq = seq + lax.shift_right_logical(seq, 8)
                v = plsc.bitcast(plsc.load_gather(lg, [seq]), jnp.int32)
                m = lax.shift_right_arithmetic(v, 31)
                k = v ^ ((m ^ jnp.int32(-1)) & jnp.int32(0x7FFFFFFF))
                plsc.store_scatter(ka, [padseq], k)
                plsc.store_scatter(pa, [padseq], seq)
            return 0
        lax.fori_loop(0, nvec // _UNROLL, mk, 0)

        def radix_pass(src_k, src_p, dst_k, dst_p, shift, last=False):
            def z(jo, _):
                for u in range(_UNROLL):
                    plsc.store_scatter(
                        hist, [(jo * _UNROLL + u) * _L + lane], zero16)
                return 0
            lax.fori_loop(0, _HIST_WORDS // (_L * _UNROLL), z, 0)

            def h(jo, _):
                for u in range(_UNROLL):
                    j = jo * _UNROLL + u
                    k = plsc.load_gather(src_k, [lane_blk + j])
                    d = lax.shift_right_logical(k, shift) & 255
                    plsc.addupdate_scatter(hist, [lane_hist + d], ones16)
                return 0
            lax.fori_loop(0, nvec // _UNROLL, h, 0)

            def b1(c, _):
                dchunk = c * _L + lane
                s = zero16
                for l in range(_L):
                    hl = plsc.load_gather(hist, [l * (_NBIN + 1) + dchunk])
                    plsc.store_scatter(hist, [l * (_NBIN + 1) + dchunk], s)
                    s = s + hl
                plsc.store_scatter(tot, [dchunk], s)
                return 0
            lax.fori_loop(0, _NBIN // _L, b1, 0)

            def b2(c, carry):
                v = plsc.load_gather(tot, [c * _L + lane])
                incl = plsc.cumsum(v)
                plsc.store_scatter(dbase, [c * _L + lane],
                                   (incl - v) + carry)
                return carry + jnp.sum(v)
            lax.fori_loop(0, _NBIN // _L, b2, jnp.int32(0))

            def b3(c, _):
                dchunk = c * _L + lane
                bv = plsc.load_gather(dbase, [dchunk])
                for l in range(_L):
                    p = plsc.load_gather(hist, [l * (_NBIN + 1) + dchunk])
                    plsc.store_scatter(hist, [l * (_NBIN + 1) + dchunk],
                                       p + bv)
                return 0
            lax.fori_loop(0, _NBIN // _L, b3, 0)

            def sc(jo, _):
                for u in range(_UNROLL):
                    j = jo * _UNROLL + u
                    k = plsc.load_gather(src_k, [lane_blk + j])
                    p = plsc.load_gather(src_p, [lane_blk + j])
                    d = lax.shift_right_logical(k, shift) & 255
                    hidx = lane_hist + d
                    dest = plsc.load_gather(hist, [hidx])
                    plsc.addupdate_scatter(hist, [hidx], ones16)
                    if last:
                        m2 = lax.shift_right_arithmetic(k, 31)
                        v = plsc.bitcast(
                            k ^ ((m2 ^ jnp.int32(-1)) & jnp.int32(0x7FFFFFFF)),
                            jnp.float32)
                        s = 1.0 / (1.0 + jnp.exp(-v))
                        plsc.store_scatter(sg, [dest], s)
                        plsc.store_scatter(dst_p, [dest], p)
                    else:
                        pdest = dest + lax.shift_right_logical(dest, 8)
                        plsc.store_scatter(dst_k, [pdest], k)
                        plsc.store_scatter(dst_p, [pdest], p)
                return 0
            lax.fori_loop(0, nvec // _UNROLL, sc, 0)

        radix_pass(ka, pa, kb, pb, 0)
        radix_pass(kb, pb, ka, pa, 8)
        radix_pass(ka, pa, kb, pb, 16)
        radix_pass(kb, pb, ka, pa, 24, last=True)

        pltpu.sync_copy(sg, sig_hbm.at[pl.ds(row, t)])
        pltpu.sync_copy(pa.at[pl.ds(0, t)], idx_hbm.at[pl.ds(row, t)])


def kernel(x, W, capacity_factor):
    B, T, C = x.shape
    TT = min(2048, T)

    logits = pl.pallas_call(
        _logits_body,
        grid=(B, T // TT),
        in_specs=[
            pl.BlockSpec((1, TT, C), lambda b, t: (b, t, 0)),
            pl.BlockSpec((1, C), lambda b, t: (0, 0)),
        ],
        out_specs=pl.BlockSpec((1, 1, TT), lambda b, t: (b, 0, t)),
        out_shape=jax.ShapeDtypeStruct((B, 1, T), jnp.float32),
    )(x, W)

    mesh = plsc.VectorSubcoreMesh(core_axis_name="c", subcore_axis_name="s")
    sc_sort = functools.partial(
        pl.kernel,
        mesh=mesh,
        out_type=[
            jax.ShapeDtypeStruct((B * T,), jnp.float32),
            jax.ShapeDtypeStruct((B * T,), jnp.int32),
        ],
        scratch_types=[
            pltpu.VMEM((T,), jnp.float32),
            pltpu.VMEM((T + _L,), jnp.int32),
            pltpu.VMEM((T + _L,), jnp.int32),
            pltpu.VMEM((T + _L,), jnp.int32),
            pltpu.VMEM((T + _L,), jnp.int32),
            pltpu.VMEM((_HIST_WORDS,), jnp.int32),
            pltpu.VMEM((_NBIN,), jnp.int32),
            pltpu.VMEM((_NBIN,), jnp.int32),
            pltpu.VMEM((T,), jnp.float32),
        ],
        compiler_params=pltpu.CompilerParams(
            use_tc_tiling_on_sc=False, needs_layout_passes=False),
    )(functools.partial(_sc_sort_body, B, T))

    sig_flat, idx_flat = sc_sort(logits.reshape(B * T))

    weights = sig_flat.reshape(B, T, 1)
    selected_tokens = idx_flat.reshape(B, T, 1)
    is_final = jnp.zeros((B, T), dtype=bool)
    return (is_final, selected_tokens, weights)


# speedup vs baseline: 1.0862x; 1.0051x over previous
import functools

import jax
import jax.numpy as jnp
from jax import lax
from jax.experimental import pallas as pl
from jax.experimental.pallas import tpu as pltpu
from jax.experimental.pallas import tpu_sc as plsc

_L = 16
_NBIN = 256
_UNROLL = 8
_HIST_WORDS = -(-(16 * 257) // (16 * _UNROLL)) * (16 * _UNROLL)


def _logits_body(x_ref, w_ref, out_ref):
    xb = x_ref[0]
    w = w_ref[...]
    out_ref[0] = jax.lax.dot_general(
        w, xb, (((1,), (1,)), ((), ())),
        preferred_element_type=jnp.float32)


def _sc_sort_body(nbatch, t, logit_hbm, sig_hbm, idx_hbm,
                  lg, ka, kb, pa, pb, hist, tot, dbase, sg):
    lane = lax.iota(jnp.int32, _L)
    per_lane = t // _L
    _PL = per_lane + 1
    lane_blk = lane * _PL
    lane_hist = lane * (_NBIN + 1)
    zero16 = jnp.zeros((_L,), jnp.int32)
    ones16 = jnp.ones((_L,), jnp.int32)
    nvec = t // _L

    wid = lax.axis_index("s") * 2 + lax.axis_index("c")

    @pl.when(wid < nbatch)
    def _():
        row = wid * t
        pltpu.sync_copy(logit_hbm.at[pl.ds(row, t)], lg)

        def mk(jo, _):
            for u in range(_UNROLL):
                j = jo * _UNROLL + u
                seq = j * _L + lane
                padseq = seq + lax.shift_right_logical(seq, 8)
                v = plsc.bitcast(lg[pl.ds(j * _L, _L)], jnp.int32)
                m = lax.shift_right_arithmetic(v, 31)
                k = v ^ ((m ^ jnp.int32(-1)) & jnp.int32(0x7FFFFFFF))
                plsc.store_scatter(ka, [padseq], k)
                plsc.store_scatter(pa, [padseq], seq)
            return 0
        lax.fori_loop(0, nvec // _UNROLL, mk, 0)

        def radix_pass(src_k, src_p, dst_k, dst_p, shift, last=False):
            def z(jo, _):
                for u in range(_UNROLL):
                    hist[pl.ds((jo * _UNROLL + u) * _L, _L)] = zero16
                return 0
            lax.fori_loop(0, _HIST_WORDS // (_L * _UNROLL), z, 0)

            def h(jo, _):
                for u in range(_UNROLL):
                    j = jo * _UNROLL + u
                    k = plsc.load_gather(src_k, [lane_blk + j])
                    d = lax.shift_right_logical(k, shift) & 255
                    plsc.addupdate_scatter(hist, [lane_hist + d], ones16)
                return 0
            lax.fori_loop(0, nvec // _UNROLL, h, 0)

            def b1(c, _):
                dchunk = c * _L + lane
                s = zero16
                for l in range(_L):
                    hl = plsc.load_gather(hist, [l * (_NBIN + 1) + dchunk])
                    plsc.store_scatter(hist, [l * (_NBIN + 1) + dchunk], s)
                    s = s + hl
                plsc.store_scatter(tot, [dchunk], s)
                return 0
            lax.fori_loop(0, _NBIN // _L, b1, 0)

            def b2(c, carry):
                v = plsc.load_gather(tot, [c * _L + lane])
                incl = plsc.cumsum(v)
                plsc.store_scatter(dbase, [c * _L + lane],
                                   (incl - v) + carry)
                return carry + jnp.sum(v)
            lax.fori_loop(0, _NBIN // _L, b2, jnp.int32(0))

            def b3(c, _):
                dchunk = c * _L + lane
                bv = plsc.load_gather(dbase, [dchunk])
                for l in range(_L):
                    p = plsc.load_gather(hist, [l * (_NBIN + 1) + dchunk])
                    plsc.store_scatter(hist, [l * (_NBIN + 1) + dchunk],
                                       p + bv)
                return 0
            lax.fori_loop(0, _NBIN // _L, b3, 0)

            def sc(jo, _):
                for u in range(_UNROLL):
                    j = jo * _UNROLL + u
                    k = plsc.load_gather(src_k, [lane_blk + j])
                    p = plsc.load_gather(src_p, [lane_blk + j])
                    d = lax.shift_right_logical(k, shift) & 255
                    hidx = lane_hist + d
                    dest = plsc.load_gather(hist, [hidx])
                    plsc.addupdate_scatter(hist, [hidx], ones16)
                    if last:
                        m2 = lax.shift_right_arithmetic(k, 31)
                        v = plsc.bitcast(
                            k ^ ((m2 ^ jnp.int32(-1)) & jnp.int32(0x7FFFFFFF)),
                            jnp.float32)
                        s = 1.0 / (1.0 + jnp.exp(-v))
                        plsc.store_scatter(sg, [dest], s)
                        plsc.store_scatter(dst_p, [dest], p)
                    else:
                        pdest = dest + lax.shift_right_logical(dest, 8)
                        plsc.store_scatter(dst_k, [pdest], k)
                        plsc.store_scatter(dst_p, [pdest], p)
                return 0
            lax.fori_loop(0, nvec // _UNROLL, sc, 0)

        radix_pass(ka, pa, kb, pb, 0)
        radix_pass(kb, pb, ka, pa, 8)
        radix_pass(ka, pa, kb, pb, 16)
        radix_pass(kb, pb, ka, pa, 24, last=True)

        pltpu.sync_copy(sg, sig_hbm.at[pl.ds(row, t)])
        pltpu.sync_copy(pa.at[pl.ds(0, t)], idx_hbm.at[pl.ds(row, t)])


def kernel(x, W, capacity_factor):
    B, T, C = x.shape
    TT = min(2048, T)

    logits = pl.pallas_call(
        _logits_body,
        grid=(B, T // TT),
        in_specs=[
            pl.BlockSpec((1, TT, C), lambda b, t: (b, t, 0)),
            pl.BlockSpec((1, C), lambda b, t: (0, 0)),
        ],
        out_specs=pl.BlockSpec((1, 1, TT), lambda b, t: (b, 0, t)),
        out_shape=jax.ShapeDtypeStruct((B, 1, T), jnp.float32),
    )(x, W)

    mesh = plsc.VectorSubcoreMesh(core_axis_name="c", subcore_axis_name="s")
    sc_sort = functools.partial(
        pl.kernel,
        mesh=mesh,
        out_type=[
            jax.ShapeDtypeStruct((B * T,), jnp.float32),
            jax.ShapeDtypeStruct((B * T,), jnp.int32),
        ],
        scratch_types=[
            pltpu.VMEM((T,), jnp.float32),
            pltpu.VMEM((T + _L,), jnp.int32),
            pltpu.VMEM((T + _L,), jnp.int32),
            pltpu.VMEM((T + _L,), jnp.int32),
            pltpu.VMEM((T + _L,), jnp.int32),
            pltpu.VMEM((_HIST_WORDS,), jnp.int32),
            pltpu.VMEM((_NBIN,), jnp.int32),
            pltpu.VMEM((_NBIN,), jnp.int32),
            pltpu.VMEM((T,), jnp.float32),
        ],
        compiler_params=pltpu.CompilerParams(
            use_tc_tiling_on_sc=False, needs_layout_passes=False),
    )(functools.partial(_sc_sort_body, B, T))

    sig_flat, idx_flat = sc_sort(logits.reshape(B * T))

    weights = sig_flat.reshape(B, T, 1)
    selected_tokens = idx_flat.reshape(B, T, 1)
    is_final = jnp.zeros((B, T), dtype=bool)
    return (is_final, selected_tokens, weights)


# speedup vs baseline: 1.1735x; 1.0803x over previous
import functools

import jax
import jax.numpy as jnp
from jax import lax
from jax.experimental import pallas as pl
from jax.experimental.pallas import tpu as pltpu
from jax.experimental.pallas import tpu_sc as plsc

_L = 16
_NBIN = 256
_UNROLL = 8
_HIST_WORDS = -(-(16 * 257) // (16 * _UNROLL)) * (16 * _UNROLL)


def _logits_body(x_ref, w_ref, out_ref):
    xb = x_ref[0]
    w = w_ref[...]
    out_ref[0] = jax.lax.dot_general(
        w, xb, (((1,), (1,)), ((), ())),
        preferred_element_type=jnp.float32)


def _sc_sort_body(nbatch, t, logit_hbm, sig_hbm, idx_hbm,
                  lg, ka, kb, pa, pb, hist, tot, dbase, sg):
    lane = lax.iota(jnp.int32, _L)
    per_lane = t // _L
    _PL = per_lane + 1
    lane_blk = lane * _PL
    lane_hist = lane * (_NBIN + 1)
    zero16 = jnp.zeros((_L,), jnp.int32)
    ones16 = jnp.ones((_L,), jnp.int32)
    nvec = t // _L

    wid = lax.axis_index("s") * 2 + lax.axis_index("c")

    @pl.when(wid < nbatch)
    def _():
        row = wid * t
        pltpu.sync_copy(logit_hbm.at[pl.ds(row, t)], lg)

        def mk(jo, _):
            for u in range(_UNROLL):
                j = jo * _UNROLL + u
                seq = j * _L + lane
                padseq = seq + lax.shift_right_logical(seq, 8)
                v = plsc.bitcast(lg[pl.ds(j * _L, _L)], jnp.int32)
                m = lax.shift_right_arithmetic(v, 31)
                k = v ^ ((m ^ jnp.int32(-1)) & jnp.int32(0x7FFFFFFF))
                plsc.store_scatter(ka, [padseq], k)
                plsc.store_scatter(pa, [padseq], seq)
            return 0
        lax.fori_loop(0, nvec // _UNROLL, mk, 0)

        def radix_pass(src_k, src_p, dst_k, dst_p, shift, last=False):
            def z(jo, _):
                for u in range(_UNROLL):
                    hist[pl.ds((jo * _UNROLL + u) * _L, _L)] = zero16
                return 0
            lax.fori_loop(0, _HIST_WORDS // (_L * _UNROLL), z, 0)

            def h(jo, _):
                for u in range(_UNROLL):
                    j = jo * _UNROLL + u
                    k = plsc.load_gather(src_k, [lane_blk + j])
                    d = lax.shift_right_logical(k, shift) & 255
                    plsc.addupdate_scatter(hist, [lane_hist + d], ones16)
                return 0
            lax.fori_loop(0, nvec // _UNROLL, h, 0)

            def b1(c, _):
                s = zero16
                for l in range(_L):
                    sl = pl.ds(l * (_NBIN + 1) + c * _L, _L)
                    hl = hist[sl]
                    hist[sl] = s
                    s = s + hl
                tot[pl.ds(c * _L, _L)] = s
                return 0
            lax.fori_loop(0, _NBIN // _L, b1, 0)

            def b2(c, carry):
                v = tot[pl.ds(c * _L, _L)]
                incl = plsc.cumsum(v)
                dbase[pl.ds(c * _L, _L)] = (incl - v) + carry
                return carry + jnp.sum(v)
            lax.fori_loop(0, _NBIN // _L, b2, jnp.int32(0))

            def b3(c, _):
                bv = dbase[pl.ds(c * _L, _L)]
                for l in range(_L):
                    sl = pl.ds(l * (_NBIN + 1) + c * _L, _L)
                    hist[sl] = hist[sl] + bv
                return 0
            lax.fori_loop(0, _NBIN // _L, b3, 0)

            def sc(jo, _):
                for u in range(_UNROLL):
                    j = jo * _UNROLL + u
                    k = plsc.load_gather(src_k, [lane_blk + j])
                    p = plsc.load_gather(src_p, [lane_blk + j])
                    d = lax.shift_right_logical(k, shift) & 255
                    hidx = lane_hist + d
                    dest = plsc.load_gather(hist, [hidx])
                    plsc.addupdate_scatter(hist, [hidx], ones16)
                    if last:
                        m2 = lax.shift_right_arithmetic(k, 31)
                        v = plsc.bitcast(
                            k ^ ((m2 ^ jnp.int32(-1)) & jnp.int32(0x7FFFFFFF)),
                            jnp.float32)
                        s = 1.0 / (1.0 + jnp.exp(-v))
                        plsc.store_scatter(sg, [dest], s)
                        plsc.store_scatter(dst_p, [dest], p)
                    else:
                        pdest = dest + lax.shift_right_logical(dest, 8)
                        plsc.store_scatter(dst_k, [pdest], k)
                        plsc.store_scatter(dst_p, [pdest], p)
                return 0
            lax.fori_loop(0, nvec // _UNROLL, sc, 0)

        radix_pass(ka, pa, kb, pb, 0)
        radix_pass(kb, pb, ka, pa, 8)
        radix_pass(ka, pa, kb, pb, 16)
        radix_pass(kb, pb, ka, pa, 24, last=True)

        pltpu.sync_copy(sg, sig_hbm.at[pl.ds(row, t)])
        pltpu.sync_copy(pa.at[pl.ds(0, t)], idx_hbm.at[pl.ds(row, t)])


def kernel(x, W, capacity_factor):
    B, T, C = x.shape
    TT = min(2048, T)

    logits = pl.pallas_call(
        _logits_body,
        grid=(B, T // TT),
        in_specs=[
            pl.BlockSpec((1, TT, C), lambda b, t: (b, t, 0)),
            pl.BlockSpec((1, C), lambda b, t: (0, 0)),
        ],
        out_specs=pl.BlockSpec((1, 1, TT), lambda b, t: (b, 0, t)),
        out_shape=jax.ShapeDtypeStruct((B, 1, T), jnp.float32),
    )(x, W)

    mesh = plsc.VectorSubcoreMesh(core_axis_name="c", subcore_axis_name="s")
    sc_sort = functools.partial(
        pl.kernel,
        mesh=mesh,
        out_type=[
            jax.ShapeDtypeStruct((B * T,), jnp.float32),
            jax.ShapeDtypeStruct((B * T,), jnp.int32),
        ],
        scratch_types=[
            pltpu.VMEM((T,), jnp.float32),
            pltpu.VMEM((T + _L,), jnp.int32),
            pltpu.VMEM((T + _L,), jnp.int32),
            pltpu.VMEM((T + _L,), jnp.int32),
            pltpu.VMEM((T + _L,), jnp.int32),
            pltpu.VMEM((_HIST_WORDS,), jnp.int32),
            pltpu.VMEM((_NBIN,), jnp.int32),
            pltpu.VMEM((_NBIN,), jnp.int32),
            pltpu.VMEM((T,), jnp.float32),
        ],
        compiler_params=pltpu.CompilerParams(
            use_tc_tiling_on_sc=False, needs_layout_passes=False),
    )(functools.partial(_sc_sort_body, B, T))

    sig_flat, idx_flat = sc_sort(logits.reshape(B * T))

    weights = sig_flat.reshape(B, T, 1)
    selected_tokens = idx_flat.reshape(B, T, 1)
    is_final = jnp.zeros((B, T), dtype=bool)
    return (is_final, selected_tokens, weights)


# speedup vs baseline: 1.1784x; 1.0041x over previous
import functools

import jax
import jax.numpy as jnp
from jax import lax
from jax.experimental import pallas as pl
from jax.experimental.pallas import tpu as pltpu
from jax.experimental.pallas import tpu_sc as plsc

_L = 16
_NBIN = 256
_UNROLL = 8
_HIST_WORDS = -(-(16 * 257) // (16 * _UNROLL)) * (16 * _UNROLL)


def _logits_body(x_ref, w_ref, out_ref):
    xb = x_ref[0]
    w = w_ref[...]
    out_ref[0] = jax.lax.dot_general(
        w, xb, (((1,), (1,)), ((), ())),
        preferred_element_type=jnp.float32)


def _sc_sort_body(nbatch, t, logit_hbm, sig_hbm, idx_hbm,
                  lg, ka, kb, pa, pb, hist, tot, dbase, sg):
    lane = lax.iota(jnp.int32, _L)
    per_lane = t // _L
    _PL = per_lane + 1
    lane_blk = lane * _PL
    lane_hist = lane * (_NBIN + 1)
    zero16 = jnp.zeros((_L,), jnp.int32)
    ones16 = jnp.ones((_L,), jnp.int32)
    nvec = t // _L

    wid = lax.axis_index("s") * 2 + lax.axis_index("c")

    @pl.when(wid < nbatch)
    def _():
        row = wid * t
        pltpu.sync_copy(logit_hbm.at[pl.ds(row, t)], lg)

        def mk(jo, _):
            for u in range(_UNROLL):
                j = jo * _UNROLL + u
                seq = j * _L + lane
                pad = pl.ds(j * _L + (j // _L), _L)
                v = plsc.bitcast(lg[pl.ds(j * _L, _L)], jnp.int32)
                m = lax.shift_right_arithmetic(v, 31)
                k = v ^ ((m ^ jnp.int32(-1)) & jnp.int32(0x7FFFFFFF))
                ka[pad] = k
                pa[pad] = seq
            return 0
        lax.fori_loop(0, nvec // _UNROLL, mk, 0)

        def radix_pass(src_k, src_p, dst_k, dst_p, shift, last=False):
            def z(jo, _):
                for u in range(_UNROLL):
                    hist[pl.ds((jo * _UNROLL + u) * _L, _L)] = zero16
                return 0
            lax.fori_loop(0, _HIST_WORDS // (_L * _UNROLL), z, 0)

            def h(jo, _):
                for u in range(_UNROLL):
                    j = jo * _UNROLL + u
                    k = plsc.load_gather(src_k, [lane_blk + j])
                    d = lax.shift_right_logical(k, shift) & 255
                    plsc.addupdate_scatter(hist, [lane_hist + d], ones16)
                return 0
            lax.fori_loop(0, nvec // _UNROLL, h, 0)

            def b1(c, _):
                s = zero16
                for l in range(_L):
                    sl = pl.ds(l * (_NBIN + 1) + c * _L, _L)
                    hl = hist[sl]
                    hist[sl] = s
                    s = s + hl
                tot[pl.ds(c * _L, _L)] = s
                return 0
            lax.fori_loop(0, _NBIN // _L, b1, 0)

            def b2(c, carry):
                v = tot[pl.ds(c * _L, _L)]
                incl = plsc.cumsum(v)
                dbase[pl.ds(c * _L, _L)] = (incl - v) + carry
                return carry + jnp.sum(v)
            lax.fori_loop(0, _NBIN // _L, b2, jnp.int32(0))

            def b3(c, _):
                bv = dbase[pl.ds(c * _L, _L)]
                for l in range(_L):
                    sl = pl.ds(l * (_NBIN + 1) + c * _L, _L)
                    hist[sl] = hist[sl] + bv
                return 0
            lax.fori_loop(0, _NBIN // _L, b3, 0)

            def sc(jo, _):
                for u in range(_UNROLL):
                    j = jo * _UNROLL + u
                    k = plsc.load_gather(src_k, [lane_blk + j])
                    p = plsc.load_gather(src_p, [lane_blk + j])
                    d = lax.shift_right_logical(k, shift) & 255
                    hidx = lane_hist + d
                    dest = plsc.load_gather(hist, [hidx])
                    plsc.addupdate_scatter(hist, [hidx], ones16)
                    if last:
                        m2 = lax.shift_right_arithmetic(k, 31)
                        v = plsc.bitcast(
                            k ^ ((m2 ^ jnp.int32(-1)) & jnp.int32(0x7FFFFFFF)),
                            jnp.float32)
                        s = 1.0 / (1.0 + jnp.exp(-v))
                        plsc.store_scatter(sg, [dest], s)
                        plsc.store_scatter(dst_p, [dest], p)
                    else:
                        pdest = dest + lax.shift_right_logical(dest, 8)
                        plsc.store_scatter(dst_k, [pdest], k)
                        plsc.store_scatter(dst_p, [pdest], p)
                return 0
            lax.fori_loop(0, nvec // _UNROLL, sc, 0)

        radix_pass(ka, pa, kb, pb, 0)
        radix_pass(kb, pb, ka, pa, 8)
        radix_pass(ka, pa, kb, pb, 16)
        radix_pass(kb, pb, ka, pa, 24, last=True)

        pltpu.sync_copy(sg, sig_hbm.at[pl.ds(row, t)])
        pltpu.sync_copy(pa.at[pl.ds(0, t)], idx_hbm.at[pl.ds(row, t)])


def kernel(x, W, capacity_factor):
    B, T, C = x.shape
    TT = min(2048, T)

    logits = pl.pallas_call(
        _logits_body,
        grid=(B, T // TT),
        in_specs=[
            pl.BlockSpec((1, TT, C), lambda b, t: (b, t, 0)),
            pl.BlockSpec((1, C), lambda b, t: (0, 0)),
        ],
        out_specs=pl.BlockSpec((1, 1, TT), lambda b, t: (b, 0, t)),
        out_shape=jax.ShapeDtypeStruct((B, 1, T), jnp.float32),
    )(x, W)

    mesh = plsc.VectorSubcoreMesh(core_axis_name="c", subcore_axis_name="s")
    sc_sort = functools.partial(
        pl.kernel,
        mesh=mesh,
        out_type=[
            jax.ShapeDtypeStruct((B * T,), jnp.float32),
            jax.ShapeDtypeStruct((B * T,), jnp.int32),
        ],
        scratch_types=[
            pltpu.VMEM((T,), jnp.float32),
            pltpu.VMEM((T + _L,), jnp.int32),
            pltpu.VMEM((T + _L,), jnp.int32),
            pltpu.VMEM((T + _L,), jnp.int32),
            pltpu.VMEM((T + _L,), jnp.int32),
            pltpu.VMEM((_HIST_WORDS,), jnp.int32),
            pltpu.VMEM((_NBIN,), jnp.int32),
            pltpu.VMEM((_NBIN,), jnp.int32),
            pltpu.VMEM((T,), jnp.float32),
        ],
        compiler_params=pltpu.CompilerParams(
            use_tc_tiling_on_sc=False, needs_layout_passes=False),
    )(functools.partial(_sc_sort_body, B, T))

    sig_flat, idx_flat = sc_sort(logits.reshape(B * T))

    weights = sig_flat.reshape(B, T, 1)
    selected_tokens = idx_flat.reshape(B, T, 1)
    is_final = jnp.zeros((B, T), dtype=bool)
    return (is_final, selected_tokens, weights)
